# Initial kernel scaffold; baseline (speedup 1.0000x reference)
import jax
import jax.numpy as jnp
from jax.experimental import pallas as pl


def kernel(doc_codes, sub_weights):
    raise NotImplementedError("write your pallas kernel here")


# speedup vs baseline: 20.0289x; 20.0289x over previous
import functools

import jax
import jax.numpy as jnp
from jax import lax
from jax.experimental import pallas as pl
from jax.experimental.pallas import tpu as pltpu
from jax.experimental.pallas import tpu_sc as plsc

_M = 48
_K = 256
_DSUB = 16
_B = 16384

_NC = 2
_NS = 16
_NW = _NC * _NS
_TOTAL = _B * _M
_PER_W = _TOTAL // _NW
_RPG = 128
_NG = _PER_W // _RPG
_KF = 8
_NB = _NG // _KF

_mesh = plsc.VectorSubcoreMesh(core_axis_name="c", subcore_axis_name="s")


@functools.partial(
    pl.kernel,
    mesh=_mesh,
    out_type=jax.ShapeDtypeStruct((_TOTAL, _DSUB), jnp.float32),
    scratch_types=[
        pltpu.VMEM((_NG, _RPG), jnp.int32),
        pltpu.VMEM((_KF * _RPG, _DSUB), jnp.float32),
        pltpu.SemaphoreType.DMA,
    ],
    compiler_params=pltpu.CompilerParams(use_tc_tiling_on_sc=False),
)
def _pq_gather(codes_hbm, table_hbm, out_hbm, idx_v, rows_v, sem):
    wid = lax.axis_index("s") * _NC + lax.axis_index("c")

    pltpu.sync_copy(codes_hbm.at[pl.ds(wid * _NG, _NG)], idx_v)

    lane = lax.iota(jnp.int32, 16)

    def add_offsets(j, carry):
        for o in range(_RPG // 16):
            pos = j * _RPG + (o * 16) + lane
            off = lax.rem(pos, _M) * _K
            sl = pl.ds(o * 16, 16)
            idx_v[j, sl] = idx_v[j, sl] + off
        return carry

    lax.fori_loop(0, _NG, add_offsets, 0)

    def burst(g, carry):
        copies = []
        for f in range(_KF):
            copies.append(
                pltpu.async_copy(
                    table_hbm.at[idx_v.at[g * _KF + f]],
                    rows_v.at[pl.ds(f * _RPG, _RPG)],
                    sem,
                )
            )
        for c in copies:
            c.wait()
        base = wid * _PER_W + g * (_KF * _RPG)
        pltpu.sync_copy(rows_v, out_hbm.at[pl.ds(base, _KF * _RPG)])
        return carry

    lax.fori_loop(0, _NB, burst, 0)


def kernel(doc_codes, sub_weights):
    codes = doc_codes.astype(jnp.int32).reshape(_NW * _NG, _RPG)
    table = sub_weights.reshape(_M * _K, _DSUB)
    out = _pq_gather(codes, table)
    return out.reshape(_B, _M * _DSUB)


# speedup vs baseline: 20.3924x; 1.0181x over previous
import functools

import jax
import jax.numpy as jnp
from jax import lax
from jax.experimental import pallas as pl
from jax.experimental.pallas import tpu as pltpu
from jax.experimental.pallas import tpu_sc as plsc

_M = 48
_K = 256
_DSUB = 16
_B = 16384

_NC = 2
_NS = 16
_NW = _NC * _NS
_TOTAL = _B * _M
_PER_W = _TOTAL // _NW
_RPG = 128
_NG = _PER_W // _RPG
_KF = 8
_NB = _NG // _KF

_mesh = plsc.VectorSubcoreMesh(core_axis_name="c", subcore_axis_name="s")


@functools.partial(
    pl.kernel,
    mesh=_mesh,
    out_type=jax.ShapeDtypeStruct((_TOTAL, _DSUB), jnp.float32),
    scratch_types=[
        pltpu.VMEM((_NG, _RPG), jnp.int32),
        pltpu.VMEM((2, _KF * _RPG, _DSUB), jnp.float32),
        pltpu.SemaphoreType.DMA,
        pltpu.SemaphoreType.DMA,
    ],
    compiler_params=pltpu.CompilerParams(use_tc_tiling_on_sc=False),
)
def _pq_gather(codes_hbm, table_hbm, out_hbm, idx_v, rows_v, sem_g, sem_s):
    wid = lax.axis_index("s") * _NC + lax.axis_index("c")

    pltpu.sync_copy(codes_hbm.at[pl.ds(wid * _NG, _NG)], idx_v)

    lane = lax.iota(jnp.int32, 16)

    def add_offsets(j, carry):
        for o in range(_RPG // 16):
            pos = j * _RPG + (o * 16) + lane
            off = lax.rem(pos, _M) * _K
            sl = pl.ds(o * 16, 16)
            idx_v[j, sl] = idx_v[j, sl] + off
        return carry

    lax.fori_loop(0, _NG, add_offsets, 0)

    _BURST = _KF * _RPG

    def burst_pair(i, carry):
        for b2 in range(2):
            g = 2 * i + b2

            @pl.when(g >= 2)
            def _drain_prev():
                pltpu.make_async_copy(
                    rows_v.at[b2],
                    out_hbm.at[pl.ds(wid * _PER_W, _BURST)],
                    sem_s,
                ).wait()

            copies = []
            for f in range(_KF):
                copies.append(
                    pltpu.async_copy(
                        table_hbm.at[idx_v.at[g * _KF + f]],
                        rows_v.at[b2, pl.ds(f * _RPG, _RPG)],
                        sem_g,
                    )
                )
            for c in copies:
                c.wait()
            pltpu.async_copy(
                rows_v.at[b2],
                out_hbm.at[pl.ds(wid * _PER_W + g * _BURST, _BURST)],
                sem_s,
            )
        return carry

    lax.fori_loop(0, _NB // 2, burst_pair, 0)

    for b2 in range(2):
        pltpu.make_async_copy(
            rows_v.at[b2],
            out_hbm.at[pl.ds(wid * _PER_W, _BURST)],
            sem_s,
        ).wait()


def kernel(doc_codes, sub_weights):
    codes = doc_codes.astype(jnp.int32).reshape(_NW * _NG, _RPG)
    table = sub_weights.reshape(_M * _K, _DSUB)
    out = _pq_gather(codes, table)
    return out.reshape(_B, _M * _DSUB)
